# baseline (speedup 1.0000x reference)
import functools

import jax
import jax.numpy as jnp
from jax import lax
from jax.experimental import pallas as pl
from jax.experimental.pallas import tpu as pltpu
from jax.experimental.pallas import tpu_sc as plsc

B = 2
S = 4096
D = 1024
H = 16
DH = 64
DFF = 4096
R = 2
NB = 64
NHALF = 32
CHUNK = 64
NCH = S // CHUNK
BS = B * S
RBH = R * B * H
NW = 32
PER_W = RBH // NW

_f32 = jnp.float32
_i32 = jnp.int32


def _proj_body(x_ref, wqk_ref, wv_ref, rot_ref, qkv_ref, bk_ref):
    x = x_ref[...]
    qk = jnp.dot(x, wqk_ref[...], preferred_element_type=_f32)
    v = jnp.dot(x.astype(jnp.bfloat16), wv_ref[...].astype(jnp.bfloat16),
                preferred_element_type=_f32)
    rot = rot_ref[...]
    for h in range(H):
        qh = qk[:, h * DH:(h + 1) * DH]
        qkv_ref[:, h, 0:DH] = qh
        qkv_ref[:, h, DH:2 * DH] = v[:, h * DH:(h + 1) * DH]
        nh = jnp.sqrt(jnp.sum(qh * qh, axis=-1, keepdims=True)) + 1e-6
        rx = jnp.dot(qh / nh, rot, preferred_element_type=_f32)
        for r in range(R):
            g = rx[:, r * NHALF:(r + 1) * NHALF]
            iota = lax.broadcasted_iota(_i32, g.shape, 1)
            mp = jnp.max(g, axis=-1, keepdims=True)
            ip = jnp.min(jnp.where(g >= mp, iota, NB), axis=-1, keepdims=True)
            mn = jnp.max(-g, axis=-1, keepdims=True)
            inn = jnp.min(jnp.where(-g >= mn, iota, NB), axis=-1,
                          keepdims=True)
            bk = jnp.where(mp >= mn, ip, NHALF + inn)
            bk_ref[:, r * H + h:r * H + h + 1] = bk
    return


def _run_proj(x2f, wqk, wv, rotcat, bs=512):
    grid = (BS // bs,)
    return pl.pallas_call(
        _proj_body,
        grid=grid,
        in_specs=[
            pl.BlockSpec((bs, D), lambda i: (i, 0)),
            pl.BlockSpec((D, D), lambda i: (0, 0)),
            pl.BlockSpec((D, D), lambda i: (0, 0)),
            pl.BlockSpec((DH, 2 * NHALF), lambda i: (0, 0)),
        ],
        out_specs=[
            pl.BlockSpec((bs, H, 2 * DH), lambda i: (i, 0, 0)),
            pl.BlockSpec((bs, R * H), lambda i: (i, 0)),
        ],
        out_shape=[
            jax.ShapeDtypeStruct((BS, H, 2 * DH), _f32),
            jax.ShapeDtypeStruct((BS, R * H), _i32),
        ],
    )(x2f, wqk, wv, rotcat)


_CB = 512
_NCB = S // _CB


def _dest_body(bk_ref, dest_ref):
    bk = bk_ref[0]
    oh = (bk == lax.broadcasted_iota(_i32, (S, NB), 1))
    ohb = oh.astype(jnp.bfloat16)
    tril = (lax.broadcasted_iota(_i32, (_CB, _CB), 1)
            <= lax.broadcasted_iota(_i32, (_CB, _CB), 0)
            ).astype(jnp.bfloat16)
    off = jnp.zeros((1, NB), _f32)
    pieces = []
    for c in range(_NCB):
        blk = ohb[c * _CB:(c + 1) * _CB]
        incl = lax.dot_general(tril, blk, (((1,), (0,)), ((), ())),
                               preferred_element_type=_f32)
        pieces.append(incl + off)
        off = off + incl[_CB - 1:_CB]
    cums = jnp.concatenate(pieces, axis=0)
    hist = off
    s = hist
    k = 1
    while k < NB:
        s = s + jnp.concatenate(
            [jnp.zeros((1, k), _f32), s[:, :NB - k]], axis=1)
        k *= 2
    start = s - hist
    vals = cums - 1.0 + start
    dest = jnp.sum(jnp.where(oh, vals, 0.0), axis=1, keepdims=True)
    dest_ref[0] = dest.astype(_i32)
    return


def _run_dest(bk4):
    return pl.pallas_call(
        _dest_body,
        grid=(RBH,),
        in_specs=[pl.BlockSpec((1, S, 1), lambda g: (g, 0, 0))],
        out_specs=pl.BlockSpec((1, S, 1), lambda g: (g, 0, 0)),
        out_shape=jax.ShapeDtypeStruct((RBH, S, 1), _i32),
    )(bk4)


_SC_CHUNK = 512
_NCHK = S // _SC_CHUNK


def _sc_pre_body(dest_hbm, qkv_hbm, st_hbm, sqkv_hbm,
                 dest_v, st_v, idx_v, buf, sem):
    wid = lax.axis_index("s") * 2 + lax.axis_index("c")
    for t in range(PER_W):
        g = wid * PER_W + t
        b = (g % 32) // 16
        h = g % 16
        off = b * (S * H) + h
        pltpu.sync_copy(dest_hbm.at[g], dest_v)

        def sbody(i, _):
            dv = dest_v[pl.ds(i * 16, 16)]
            vals = lax.broadcasted_iota(_i32, (16,), 0) + i * 16
            plsc.store_scatter(st_v, [dv], vals)
            return 0

        lax.fori_loop(0, S // 16, sbody, 0)

        def ibody(k, _):
            st = st_v[pl.ds(k * 16, 16)]
            idx_v[pl.ds(k * 16, 16)] = st * H + off
            return 0

        lax.fori_loop(0, S // 16, ibody, 0)
        for c in range(_NCHK):
            pltpu.async_copy(
                qkv_hbm.at[idx_v.at[pl.ds(c * _SC_CHUNK, _SC_CHUNK)]],
                buf, sem).wait()
            pltpu.sync_copy(
                buf, sqkv_hbm.at[pl.ds(g * S + c * _SC_CHUNK, _SC_CHUNK)])
        pltpu.sync_copy(st_v, st_hbm.at[g])
    return


def _run_sc_pre(dest, qkv):
    mesh = plsc.VectorSubcoreMesh(core_axis_name="c", subcore_axis_name="s")
    fn = functools.partial(
        pl.kernel,
        out_type=[
            jax.ShapeDtypeStruct((RBH, S), _i32),
            jax.ShapeDtypeStruct((RBH * S, 2 * DH), _f32),
        ],
        mesh=mesh,
        compiler_params=pltpu.CompilerParams(needs_layout_passes=False),
        scratch_types=[
            pltpu.VMEM((S,), _i32),
            pltpu.VMEM((S,), _i32),
            pltpu.VMEM((S,), _i32),
            pltpu.VMEM((_SC_CHUNK, 2 * DH), _f32),
            pltpu.SemaphoreType.DMA,
        ],
    )(_sc_pre_body)
    return fn(dest, qkv)


_GC = 4
_QW = _GC * CHUNK
_KW = _QW + CHUNK
_NG = NCH // _GC


def _att_body(b_ref, a_ref, pq_ref, pk_ref, o_ref):
    qrel = lax.broadcasted_iota(_i32, (_QW, 1), 0) // CHUNK
    krel = lax.broadcasted_iota(_i32, (1, _KW), 1) // CHUNK - 1
    wmask = (krel == qrel) | (krel == qrel - 1)

    arow = a_ref[0]
    brow = b_ref[0]
    q = brow[:, 0:DH]
    kall = jnp.concatenate([arow[:, 0:DH], q], axis=0)
    vall = jnp.concatenate([arow[:, DH:2 * DH], brow[:, DH:2 * DH]],
                           axis=0)
    nrm = jnp.sqrt(jnp.sum(kall * kall, axis=-1, keepdims=True)) + 1e-6
    kn = (kall / nrm).astype(jnp.bfloat16)
    logits = lax.dot_general(
        q.astype(jnp.bfloat16), kn, (((1,), (1,)), ((), ())),
        preferred_element_type=_f32) * 0.125
    pq = pq_ref[0].astype(_f32)
    pk = pk_ref[0, 0]
    logits = jnp.where(wmask & (pq >= pk), logits, -1e9)
    m = jnp.max(logits, axis=-1, keepdims=True)
    pexp = jnp.exp(logits - m)
    ssum = jnp.sum(pexp, axis=-1, keepdims=True)
    o = lax.dot_general(
        pexp.astype(jnp.bfloat16), vall.astype(jnp.bfloat16),
        (((1,), (0,)), ((), ())), preferred_element_type=_f32) / ssum
    lse = m + jnp.log(ssum)
    o_ref[0, :, 0:DH] = o
    o_ref[0, :, DH:2 * DH] = jnp.broadcast_to(lse, (_QW, DH))
    return


def _run_att(sqkv3, pc, pkrow):
    return pl.pallas_call(
        _att_body,
        grid=(RBH, _NG),
        in_specs=[
            pl.BlockSpec((1, _QW, 2 * DH), lambda g, j: (g, j, 0)),
            pl.BlockSpec((1, CHUNK, 2 * DH),
                         lambda g, j: (g, (j * _GC + NCH - 1) % NCH, 0)),
            pl.BlockSpec((1, _QW, 1), lambda g, j: (g, j, 0)),
            pl.BlockSpec((1, 1, 1, _KW), lambda g, j: (g, j, 0, 0)),
        ],
        out_specs=pl.BlockSpec((1, _QW, 2 * DH), lambda g, j: (g, j, 0)),
        out_shape=jax.ShapeDtypeStruct((RBH, S, 2 * DH), _f32),
    )(sqkv3, sqkv3, pc, pkrow)


def _sc_post_body(dest_hbm, o_hbm, ou_hbm, dest_v, idx_v, buf, sem):
    wid = lax.axis_index("s") * 2 + lax.axis_index("c")
    for t in range(PER_W):
        g = wid * PER_W + t
        r = g // 32
        b = (g % 32) // 16
        h = g % 16
        rh = r * 16 + h
        pltpu.sync_copy(dest_hbm.at[g], dest_v)

        def ibody(k, _):
            dv = dest_v[pl.ds(k * 16, 16)]
            idx_v[pl.ds(k * 16, 16)] = dv + g * S
            return 0

        lax.fori_loop(0, S // 16, ibody, 0)
        for c in range(_NCHK):
            pltpu.async_copy(
                o_hbm.at[idx_v.at[pl.ds(c * _SC_CHUNK, _SC_CHUNK)]],
                buf, sem).wait()
            pltpu.sync_copy(
                buf,
                ou_hbm.at[pl.ds(rh * BS + b * S + c * _SC_CHUNK, _SC_CHUNK)])
    return


def _run_sc_post(dest, o2):
    mesh = plsc.VectorSubcoreMesh(core_axis_name="c", subcore_axis_name="s")
    fn = functools.partial(
        pl.kernel,
        out_type=jax.ShapeDtypeStruct((R * H * BS, 2 * DH), _f32),
        mesh=mesh,
        compiler_params=pltpu.CompilerParams(needs_layout_passes=False),
        scratch_types=[
            pltpu.VMEM((S,), _i32),
            pltpu.VMEM((S,), _i32),
            pltpu.VMEM((_SC_CHUNK, 2 * DH), _f32),
            pltpu.SemaphoreType.DMA,
        ],
    )(_sc_post_body)
    return fn(dest, o2)


def _comb_body(o3_ref, x1_ref, wo_ref, gf_ref, bf_ref, y1_ref):
    parts = []
    for h in range(H):
        r0 = o3_ref[h]
        r1 = o3_ref[H + h]
        l0 = r0[:, DH:DH + 1]
        l1 = r1[:, DH:DH + 1]
        m = jnp.maximum(l0, l1)
        e0 = jnp.exp(l0 - m)
        e1 = jnp.exp(l1 - m)
        den = e0 + e1
        oh = (r0[:, 0:DH] * (e0 / den) + r1[:, 0:DH] * (e1 / den))
        parts.append(oh)
    o = jnp.concatenate(parts, axis=1)
    f_pre = jnp.dot(o.astype(jnp.bfloat16), wo_ref[...],
                    preferred_element_type=_f32)
    mu = jnp.mean(f_pre, axis=-1, keepdims=True)
    dlt = f_pre - mu
    var = jnp.mean(dlt * dlt, axis=-1, keepdims=True)
    f = dlt * lax.rsqrt(var + 1e-12) * gf_ref[...] + bf_ref[...]
    y1_ref[...] = x1_ref[...] + f
    return


def _run_comb(o3, x1f, wo_bf, gf, bf, bs=512):
    return pl.pallas_call(
        _comb_body,
        grid=(BS // bs,),
        in_specs=[
            pl.BlockSpec((R * H, bs, 2 * DH), lambda i: (0, i, 0)),
            pl.BlockSpec((bs, D), lambda i: (i, 0)),
            pl.BlockSpec((D, D), lambda i: (0, 0)),
            pl.BlockSpec((1, D), lambda i: (0, 0)),
            pl.BlockSpec((1, D), lambda i: (0, 0)),
        ],
        out_specs=pl.BlockSpec((bs, D), lambda i: (i, 0)),
        out_shape=jax.ShapeDtypeStruct((BS, D), _f32),
    )(o3, x1f, wo_bf, gf, bf)


def _ffn_body(y1_ref, x2_ref, w1_ref, b1_ref, w2_ref, b2_ref,
              gg_ref, bg_ref, y2_ref):
    y1 = y1_ref[...]
    hpre = jnp.dot(y1.astype(jnp.bfloat16), w1_ref[...],
                   preferred_element_type=_f32) + b1_ref[...]
    hh = jnp.maximum(hpre, 0.0)
    g_pre = jnp.dot(hh.astype(jnp.bfloat16), w2_ref[...],
                    preferred_element_type=_f32) + b2_ref[...]
    mu = jnp.mean(g_pre, axis=-1, keepdims=True)
    dlt = g_pre - mu
    var = jnp.mean(dlt * dlt, axis=-1, keepdims=True)
    g = dlt * lax.rsqrt(var + 1e-12) * gg_ref[...] + bg_ref[...]
    y2_ref[...] = x2_ref[...] + g
    return


def _run_ffn(y1f, x2f, w1_bf, b1, w2_bf, b2, gg, bg, bs=512):
    return pl.pallas_call(
        _ffn_body,
        grid=(BS // bs,),
        in_specs=[
            pl.BlockSpec((bs, D), lambda i: (i, 0)),
            pl.BlockSpec((bs, D), lambda i: (i, 0)),
            pl.BlockSpec((D, DFF), lambda i: (0, 0)),
            pl.BlockSpec((1, DFF), lambda i: (0, 0)),
            pl.BlockSpec((DFF, D), lambda i: (0, 0)),
            pl.BlockSpec((1, D), lambda i: (0, 0)),
            pl.BlockSpec((1, D), lambda i: (0, 0)),
            pl.BlockSpec((1, D), lambda i: (0, 0)),
        ],
        out_specs=pl.BlockSpec((bs, D), lambda i: (i, 0)),
        out_shape=jax.ShapeDtypeStruct((BS, D), _f32),
    )(y1f, x2f, w1_bf, b1, w2_bf, b2, gg, bg)


def _sort_gather(dest, qkv):
    return _run_sc_pre(dest, qkv)


def _ungather(dest, o2):
    return _run_sc_post(dest, o2)


def kernel(x1, x2, wqk, wv, wo, gamma_f, beta_f, w1, b1, w2, b2,
           gamma_g, beta_g, rotations):
    x2f = x2.reshape(BS, D)
    x1f = x1.reshape(BS, D)
    rotcat = jnp.concatenate([rotations[0], rotations[1]], axis=1)

    qkv, buckets = _run_proj(x2f, wqk, wv, rotcat)
    qkv2 = qkv.reshape(BS * H, 2 * DH)

    bk4 = (buckets.reshape(B, S, R, H)
           .transpose(2, 0, 3, 1)
           .reshape(RBH, S, 1))
    dest3 = _run_dest(bk4)
    dest = dest3.reshape(RBH, S)

    sticker, sqkv = _sort_gather(dest, qkv2)
    sqkv3 = sqkv.reshape(RBH, S, 2 * DH)
    pc = sticker.reshape(RBH, S, 1)
    chunks = sticker.reshape(RBH, NCH, CHUNK)
    halo = jnp.roll(chunks, 1, axis=1).reshape(RBH, _NG, _GC, CHUNK)[:, :, 0]
    main = sticker.reshape(RBH, _NG, _QW)
    pkrow = jnp.concatenate([halo, main], axis=-1).astype(_f32)
    pkrow = pkrow.reshape(RBH, _NG, 1, _KW)

    o_s = _run_att(sqkv3, pc, pkrow)
    o2 = o_s.reshape(RBH * S, 2 * DH)

    o_u = _ungather(dest, o2)
    o3 = o_u.reshape(R * H, BS, 2 * DH)

    gf = gamma_f.reshape(1, D)
    bf = beta_f.reshape(1, D)
    y1f = _run_comb(o3, x1f, wo.astype(jnp.bfloat16), gf, bf)

    y2f = _run_ffn(y1f, x2f, w1.astype(jnp.bfloat16), b1.reshape(1, DFF),
                   w2.astype(jnp.bfloat16), b2.reshape(1, D),
                   gamma_g.reshape(1, D), beta_g.reshape(1, D))

    return (y1f.reshape(B, S, D), y2f.reshape(B, S, D))


# speedup vs baseline: 1.1123x; 1.1123x over previous
import functools

import jax
import jax.numpy as jnp
from jax import lax
from jax.experimental import pallas as pl
from jax.experimental.pallas import tpu as pltpu
from jax.experimental.pallas import tpu_sc as plsc

B = 2
S = 4096
D = 1024
H = 16
DH = 64
DFF = 4096
R = 2
NB = 64
NHALF = 32
CHUNK = 64
NCH = S // CHUNK
BS = B * S
RBH = R * B * H
NW = 32
PER_W = RBH // NW

_f32 = jnp.float32
_i32 = jnp.int32


def _proj_body(x_ref, wqk_ref, wv_ref, rot_ref, qkv_ref, bk_ref):
    x = x_ref[...]
    qk = jnp.dot(x, wqk_ref[...], preferred_element_type=_f32)
    v = jnp.dot(x.astype(jnp.bfloat16), wv_ref[...].astype(jnp.bfloat16),
                preferred_element_type=_f32)
    rot = rot_ref[...]
    for h in range(H):
        qh = qk[:, h * DH:(h + 1) * DH]
        qkv_ref[:, h, 0:DH] = qh
        qkv_ref[:, h, DH:2 * DH] = v[:, h * DH:(h + 1) * DH]
        nh = jnp.sqrt(jnp.sum(qh * qh, axis=-1, keepdims=True)) + 1e-6
        rx = jnp.dot(qh / nh, rot, preferred_element_type=_f32)
        for r in range(R):
            g = rx[:, r * NHALF:(r + 1) * NHALF]
            iota = lax.broadcasted_iota(_i32, g.shape, 1)
            mp = jnp.max(g, axis=-1, keepdims=True)
            ip = jnp.min(jnp.where(g >= mp, iota, NB), axis=-1, keepdims=True)
            mn = jnp.max(-g, axis=-1, keepdims=True)
            inn = jnp.min(jnp.where(-g >= mn, iota, NB), axis=-1,
                          keepdims=True)
            bk = jnp.where(mp >= mn, ip, NHALF + inn)
            bk_ref[:, r * H + h:r * H + h + 1] = bk
    return


def _run_proj(x2f, wqk, wv, rotcat, bs=512):
    grid = (BS // bs,)
    return pl.pallas_call(
        _proj_body,
        grid=grid,
        in_specs=[
            pl.BlockSpec((bs, D), lambda i: (i, 0)),
            pl.BlockSpec((D, D), lambda i: (0, 0)),
            pl.BlockSpec((D, D), lambda i: (0, 0)),
            pl.BlockSpec((DH, 2 * NHALF), lambda i: (0, 0)),
        ],
        out_specs=[
            pl.BlockSpec((bs, H, 2 * DH), lambda i: (i, 0, 0)),
            pl.BlockSpec((bs, R * H), lambda i: (i, 0)),
        ],
        out_shape=[
            jax.ShapeDtypeStruct((BS, H, 2 * DH), _f32),
            jax.ShapeDtypeStruct((BS, R * H), _i32),
        ],
    )(x2f, wqk, wv, rotcat)


_CB = 512
_NCB = S // _CB


def _dest_body(bk_ref, dest_ref):
    bk = bk_ref[0]
    oh = (bk == lax.broadcasted_iota(_i32, (S, NB), 1))
    ohb = oh.astype(jnp.bfloat16)
    tril = (lax.broadcasted_iota(_i32, (_CB, _CB), 1)
            <= lax.broadcasted_iota(_i32, (_CB, _CB), 0)
            ).astype(jnp.bfloat16)
    off = jnp.zeros((1, NB), _f32)
    pieces = []
    for c in range(_NCB):
        blk = ohb[c * _CB:(c + 1) * _CB]
        incl = lax.dot_general(tril, blk, (((1,), (0,)), ((), ())),
                               preferred_element_type=_f32)
        pieces.append(incl + off)
        off = off + incl[_CB - 1:_CB]
    cums = jnp.concatenate(pieces, axis=0)
    hist = off
    s = hist
    k = 1
    while k < NB:
        s = s + jnp.concatenate(
            [jnp.zeros((1, k), _f32), s[:, :NB - k]], axis=1)
        k *= 2
    start = s - hist
    vals = cums - 1.0 + start
    dest = jnp.sum(jnp.where(oh, vals, 0.0), axis=1, keepdims=True)
    dest_ref[0] = dest.astype(_i32)
    return


def _run_dest(bk4):
    return pl.pallas_call(
        _dest_body,
        grid=(RBH,),
        in_specs=[pl.BlockSpec((1, S, 1), lambda g: (g, 0, 0))],
        out_specs=pl.BlockSpec((1, S, 1), lambda g: (g, 0, 0)),
        out_shape=jax.ShapeDtypeStruct((RBH, S, 1), _i32),
    )(bk4)


_SC_CHUNK = 512
_NCHK = S // _SC_CHUNK


def _sc_pre_body(dest_hbm, qkv_hbm, st_hbm, sqkv_hbm,
                 dest_v, st_v, idx_v, buf, sem):
    wid = lax.axis_index("s") * 2 + lax.axis_index("c")
    for t in range(PER_W):
        g = wid * PER_W + t
        b = (g % 32) // 16
        h = g % 16
        off = b * (S * H) + h
        pltpu.sync_copy(dest_hbm.at[g], dest_v)

        def sbody(i, _):
            dv = dest_v[pl.ds(i * 16, 16)]
            vals = lax.broadcasted_iota(_i32, (16,), 0) + i * 16
            plsc.store_scatter(st_v, [dv], vals)
            return 0

        lax.fori_loop(0, S // 16, sbody, 0)

        def ibody(k, _):
            st = st_v[pl.ds(k * 16, 16)]
            idx_v[pl.ds(k * 16, 16)] = st * H + off
            return 0

        lax.fori_loop(0, S // 16, ibody, 0)
        for c in range(_NCHK):
            pltpu.async_copy(
                qkv_hbm.at[idx_v.at[pl.ds(c * _SC_CHUNK, _SC_CHUNK)]],
                buf, sem).wait()
            pltpu.sync_copy(
                buf, sqkv_hbm.at[pl.ds(g * S + c * _SC_CHUNK, _SC_CHUNK)])
        pltpu.sync_copy(st_v, st_hbm.at[g])
    return


def _run_sc_pre(dest, qkv):
    mesh = plsc.VectorSubcoreMesh(core_axis_name="c", subcore_axis_name="s")
    fn = functools.partial(
        pl.kernel,
        out_type=[
            jax.ShapeDtypeStruct((RBH, S), _i32),
            jax.ShapeDtypeStruct((RBH * S, 2 * DH), _f32),
        ],
        mesh=mesh,
        compiler_params=pltpu.CompilerParams(needs_layout_passes=False),
        scratch_types=[
            pltpu.VMEM((S,), _i32),
            pltpu.VMEM((S,), _i32),
            pltpu.VMEM((S,), _i32),
            pltpu.VMEM((_SC_CHUNK, 2 * DH), _f32),
            pltpu.SemaphoreType.DMA,
        ],
    )(_sc_pre_body)
    return fn(dest, qkv)


_GC = 8
_QW = _GC * CHUNK
_KW = _QW + CHUNK
_NG = NCH // _GC


def _att_body(b_ref, a_ref, pq_ref, pk_ref, o_ref):
    qrel = lax.broadcasted_iota(_i32, (_QW, 1), 0) // CHUNK
    krel = lax.broadcasted_iota(_i32, (1, _KW), 1) // CHUNK - 1
    wmask = (krel == qrel) | (krel == qrel - 1)

    arow = a_ref[0]
    brow = b_ref[0]
    q = brow[:, 0:DH]
    kall = jnp.concatenate([arow[:, 0:DH], q], axis=0)
    vall = jnp.concatenate([arow[:, DH:2 * DH], brow[:, DH:2 * DH]],
                           axis=0)
    nrm = jnp.sqrt(jnp.sum(kall * kall, axis=-1, keepdims=True)) + 1e-6
    kn = (kall / nrm).astype(jnp.bfloat16)
    logits = lax.dot_general(
        q.astype(jnp.bfloat16), kn, (((1,), (1,)), ((), ())),
        preferred_element_type=_f32) * 0.125
    pq = pq_ref[0].astype(_f32)
    pk = pk_ref[0, 0]
    logits = jnp.where(wmask & (pq >= pk), logits, -1e9)
    m = jnp.max(logits, axis=-1, keepdims=True)
    pexp = jnp.exp(logits - m)
    ssum = jnp.sum(pexp, axis=-1, keepdims=True)
    o = lax.dot_general(
        pexp.astype(jnp.bfloat16), vall.astype(jnp.bfloat16),
        (((1,), (0,)), ((), ())), preferred_element_type=_f32) / ssum
    lse = m + jnp.log(ssum)
    o_ref[0, :, 0:DH] = o
    o_ref[0, :, DH:2 * DH] = jnp.broadcast_to(lse, (_QW, DH))
    return


def _run_att(sqkv3, pc, pkrow):
    return pl.pallas_call(
        _att_body,
        grid=(RBH, _NG),
        in_specs=[
            pl.BlockSpec((1, _QW, 2 * DH), lambda g, j: (g, j, 0)),
            pl.BlockSpec((1, CHUNK, 2 * DH),
                         lambda g, j: (g, (j * _GC + NCH - 1) % NCH, 0)),
            pl.BlockSpec((1, _QW, 1), lambda g, j: (g, j, 0)),
            pl.BlockSpec((1, 1, 1, _KW), lambda g, j: (g, j, 0, 0)),
        ],
        out_specs=pl.BlockSpec((1, _QW, 2 * DH), lambda g, j: (g, j, 0)),
        out_shape=jax.ShapeDtypeStruct((RBH, S, 2 * DH), _f32),
    )(sqkv3, sqkv3, pc, pkrow)


def _sc_post_body(dest_hbm, o_hbm, ou_hbm, dest_v, idx_v, buf, sem):
    wid = lax.axis_index("s") * 2 + lax.axis_index("c")
    for t in range(PER_W):
        g = wid * PER_W + t
        r = g // 32
        b = (g % 32) // 16
        h = g % 16
        rh = r * 16 + h
        pltpu.sync_copy(dest_hbm.at[g], dest_v)

        def ibody(k, _):
            dv = dest_v[pl.ds(k * 16, 16)]
            idx_v[pl.ds(k * 16, 16)] = dv + g * S
            return 0

        lax.fori_loop(0, S // 16, ibody, 0)
        for c in range(_NCHK):
            pltpu.async_copy(
                o_hbm.at[idx_v.at[pl.ds(c * _SC_CHUNK, _SC_CHUNK)]],
                buf, sem).wait()
            pltpu.sync_copy(
                buf,
                ou_hbm.at[pl.ds(rh * BS + b * S + c * _SC_CHUNK, _SC_CHUNK)])
    return


def _run_sc_post(dest, o2):
    mesh = plsc.VectorSubcoreMesh(core_axis_name="c", subcore_axis_name="s")
    fn = functools.partial(
        pl.kernel,
        out_type=jax.ShapeDtypeStruct((R * H * BS, 2 * DH), _f32),
        mesh=mesh,
        compiler_params=pltpu.CompilerParams(needs_layout_passes=False),
        scratch_types=[
            pltpu.VMEM((S,), _i32),
            pltpu.VMEM((S,), _i32),
            pltpu.VMEM((_SC_CHUNK, 2 * DH), _f32),
            pltpu.SemaphoreType.DMA,
        ],
    )(_sc_post_body)
    return fn(dest, o2)


def _comb_body(o3_ref, x1_ref, wo_ref, gf_ref, bf_ref, y1_ref):
    parts = []
    for h in range(H):
        r0 = o3_ref[h]
        r1 = o3_ref[H + h]
        l0 = r0[:, DH:DH + 1]
        l1 = r1[:, DH:DH + 1]
        m = jnp.maximum(l0, l1)
        e0 = jnp.exp(l0 - m)
        e1 = jnp.exp(l1 - m)
        den = e0 + e1
        oh = (r0[:, 0:DH] * (e0 / den) + r1[:, 0:DH] * (e1 / den))
        parts.append(oh)
    o = jnp.concatenate(parts, axis=1)
    f_pre = jnp.dot(o.astype(jnp.bfloat16), wo_ref[...],
                    preferred_element_type=_f32)
    mu = jnp.mean(f_pre, axis=-1, keepdims=True)
    dlt = f_pre - mu
    var = jnp.mean(dlt * dlt, axis=-1, keepdims=True)
    f = dlt * lax.rsqrt(var + 1e-12) * gf_ref[...] + bf_ref[...]
    y1_ref[...] = x1_ref[...] + f
    return


def _run_comb(o3, x1f, wo_bf, gf, bf, bs=512):
    return pl.pallas_call(
        _comb_body,
        grid=(BS // bs,),
        in_specs=[
            pl.BlockSpec((R * H, bs, 2 * DH), lambda i: (0, i, 0)),
            pl.BlockSpec((bs, D), lambda i: (i, 0)),
            pl.BlockSpec((D, D), lambda i: (0, 0)),
            pl.BlockSpec((1, D), lambda i: (0, 0)),
            pl.BlockSpec((1, D), lambda i: (0, 0)),
        ],
        out_specs=pl.BlockSpec((bs, D), lambda i: (i, 0)),
        out_shape=jax.ShapeDtypeStruct((BS, D), _f32),
    )(o3, x1f, wo_bf, gf, bf)


def _ffn_body(y1_ref, x2_ref, w1_ref, b1_ref, w2_ref, b2_ref,
              gg_ref, bg_ref, y2_ref):
    y1 = y1_ref[...]
    hpre = jnp.dot(y1.astype(jnp.bfloat16), w1_ref[...],
                   preferred_element_type=_f32) + b1_ref[...]
    hh = jnp.maximum(hpre, 0.0)
    g_pre = jnp.dot(hh.astype(jnp.bfloat16), w2_ref[...],
                    preferred_element_type=_f32) + b2_ref[...]
    mu = jnp.mean(g_pre, axis=-1, keepdims=True)
    dlt = g_pre - mu
    var = jnp.mean(dlt * dlt, axis=-1, keepdims=True)
    g = dlt * lax.rsqrt(var + 1e-12) * gg_ref[...] + bg_ref[...]
    y2_ref[...] = x2_ref[...] + g
    return


def _run_ffn(y1f, x2f, w1_bf, b1, w2_bf, b2, gg, bg, bs=512):
    return pl.pallas_call(
        _ffn_body,
        grid=(BS // bs,),
        in_specs=[
            pl.BlockSpec((bs, D), lambda i: (i, 0)),
            pl.BlockSpec((bs, D), lambda i: (i, 0)),
            pl.BlockSpec((D, DFF), lambda i: (0, 0)),
            pl.BlockSpec((1, DFF), lambda i: (0, 0)),
            pl.BlockSpec((DFF, D), lambda i: (0, 0)),
            pl.BlockSpec((1, D), lambda i: (0, 0)),
            pl.BlockSpec((1, D), lambda i: (0, 0)),
            pl.BlockSpec((1, D), lambda i: (0, 0)),
        ],
        out_specs=pl.BlockSpec((bs, D), lambda i: (i, 0)),
        out_shape=jax.ShapeDtypeStruct((BS, D), _f32),
    )(y1f, x2f, w1_bf, b1, w2_bf, b2, gg, bg)


def _sort_gather(dest, qkv):
    return _run_sc_pre(dest, qkv)


def _ungather(dest, o2):
    return _run_sc_post(dest, o2)


def kernel(x1, x2, wqk, wv, wo, gamma_f, beta_f, w1, b1, w2, b2,
           gamma_g, beta_g, rotations):
    x2f = x2.reshape(BS, D)
    x1f = x1.reshape(BS, D)
    rotcat = jnp.concatenate([rotations[0], rotations[1]], axis=1)

    qkv, buckets = _run_proj(x2f, wqk, wv, rotcat)
    qkv2 = qkv.reshape(BS * H, 2 * DH)

    bk4 = (buckets.reshape(B, S, R, H)
           .transpose(2, 0, 3, 1)
           .reshape(RBH, S, 1))
    dest3 = _run_dest(bk4)
    dest = dest3.reshape(RBH, S)

    sticker, sqkv = _sort_gather(dest, qkv2)
    sqkv3 = sqkv.reshape(RBH, S, 2 * DH)
    pc = sticker.reshape(RBH, S, 1)
    chunks = sticker.reshape(RBH, NCH, CHUNK)
    halo = jnp.roll(chunks, 1, axis=1).reshape(RBH, _NG, _GC, CHUNK)[:, :, 0]
    main = sticker.reshape(RBH, _NG, _QW)
    pkrow = jnp.concatenate([halo, main], axis=-1).astype(_f32)
    pkrow = pkrow.reshape(RBH, _NG, 1, _KW)

    o_s = _run_att(sqkv3, pc, pkrow)
    o2 = o_s.reshape(RBH * S, 2 * DH)

    o_u = _ungather(dest, o2)
    o3 = o_u.reshape(R * H, BS, 2 * DH)

    gf = gamma_f.reshape(1, D)
    bf = beta_f.reshape(1, D)
    y1f = _run_comb(o3, x1f, wo.astype(jnp.bfloat16), gf, bf)

    y2f = _run_ffn(y1f, x2f, w1.astype(jnp.bfloat16), b1.reshape(1, DFF),
                   w2.astype(jnp.bfloat16), b2.reshape(1, D),
                   gamma_g.reshape(1, D), beta_g.reshape(1, D))

    return (y1f.reshape(B, S, D), y2f.reshape(B, S, D))
